# baseline (speedup 1.0000x reference)
import functools

import jax
import jax.numpy as jnp
from jax.experimental import pallas as pl
from jax.experimental.pallas import tpu as pltpu

E_BLOCK = 1280


def _mlp_force_body(ef_ref, hs_ref, td_ref, rx_ref, of_ref,
                    w1_ref, w2_ref, w3_ref,
                    b1_ref, g1_ref, be1_ref, b2_ref, g2_ref, be2_ref, b3_ref,
                    out_ref):
    feat = jnp.concatenate(
        [ef_ref[...], hs_ref[...] * 0.5, td_ref[...]], axis=-1)
    x = jnp.dot(feat, w1_ref[...], preferred_element_type=jnp.float32)
    x = x + b1_ref[...]
    mu = jnp.mean(x, axis=-1, keepdims=True)
    xc = x - mu
    var = jnp.mean(xc * xc, axis=-1, keepdims=True)
    x = xc * jax.lax.rsqrt(var + 1e-5) * g1_ref[...] + be1_ref[...]
    x = jnp.maximum(x, 0.0)
    x = jnp.dot(x, w2_ref[...], preferred_element_type=jnp.float32)
    x = x + b2_ref[...]
    mu = jnp.mean(x, axis=-1, keepdims=True)
    xc = x - mu
    var = jnp.mean(xc * xc, axis=-1, keepdims=True)
    x = xc * jax.lax.rsqrt(var + 1e-5) * g2_ref[...] + be2_ref[...]
    x = jnp.maximum(x, 0.0)
    pred = jnp.sum(x * w3_ref[...], axis=-1, keepdims=True) + b3_ref[...]
    rx = rx_ref[...]
    d = jnp.sqrt(jnp.sum(rx * rx, axis=-1, keepdims=True))
    scale = pred * of_ref[...] / (d * (d + 1.0))
    out_ref[...] = rx * scale


def _mlp_force(ef, hs, td, rx, of, W1, W2, w3r, b1r, g1r, be1r, b2r, g2r,
               be2r, b3r):
    E = ef.shape[0]
    grid = (E // E_BLOCK,)

    def eb(i):
        return (i, 0)

    def full(i):
        return (0, 0)

    return pl.pallas_call(
        _mlp_force_body,
        grid=grid,
        in_specs=[
            pl.BlockSpec((E_BLOCK, 16), eb),
            pl.BlockSpec((E_BLOCK, 128), eb),
            pl.BlockSpec((E_BLOCK, 16), eb),
            pl.BlockSpec((E_BLOCK, 3), eb),
            pl.BlockSpec((E_BLOCK, 1), eb),
            pl.BlockSpec((160, 256), full),
            pl.BlockSpec((256, 256), full),
            pl.BlockSpec((1, 256), full),
            pl.BlockSpec((1, 256), full),
            pl.BlockSpec((1, 256), full),
            pl.BlockSpec((1, 256), full),
            pl.BlockSpec((1, 256), full),
            pl.BlockSpec((1, 256), full),
            pl.BlockSpec((1, 256), full),
            pl.BlockSpec((1, 1), full),
        ],
        out_specs=pl.BlockSpec((E_BLOCK, 3), eb),
        out_shape=jax.ShapeDtypeStruct((E, 3), jnp.float32),
    )(ef, hs, td, rx, of, W1, W2, w3r, b1r, g1r, be1r, b2r, g2r, be2r, b3r)


def kernel(h, rel_x, edge_feat, edge_index, inner_edge_mask, t,
           W1, b1, g1, beta1, W2, b2, g2, beta2, W3, b3):
    N = h.shape[0]
    E = rel_x.shape[0]
    src = edge_index[0].astype(jnp.int32)
    dst = edge_index[1].astype(jnp.int32)

    hs = h[src] + h[dst]
    td = t[dst]

    of = (~inner_edge_mask).astype(jnp.float32).reshape(E, 1)
    w3r = W3.reshape(1, 256)
    b1r = b1.reshape(1, 256)
    g1r = g1.reshape(1, 256)
    be1r = beta1.reshape(1, 256)
    b2r = b2.reshape(1, 256)
    g2r = g2.reshape(1, 256)
    be2r = beta2.reshape(1, 256)
    b3r = b3.reshape(1, 1)

    forces = _mlp_force(edge_feat, hs, td, rel_x, of, W1, W2, w3r, b1r, g1r,
                        be1r, b2r, g2r, be2r, b3r)

    out = jnp.zeros((N, 3), dtype=jnp.float32).at[dst].add(forces)
    return out


# speedup vs baseline: 2.0170x; 2.0170x over previous
import functools

import jax
import jax.numpy as jnp
from jax import lax
from jax.experimental import pallas as pl
from jax.experimental.pallas import tpu as pltpu
from jax.experimental.pallas import tpu_sc as plsc

E_BLOCK = 1280

NC, NS = 2, 16
NW = NC * NS
GCHUNK = 80


def _gather_body(h_hbm, tp_hbm, srcr_hbm, dstr_hbm, hsum_hbm, td_hbm,
                 sidx, didx, bufh, buft, tdbuf, sem1, sem2):
    ew = srcr_hbm.shape[1] * srcr_hbm.shape[2]
    nj = srcr_hbm.shape[1]
    wid = lax.axis_index("s") * NC + lax.axis_index("c")
    pltpu.sync_copy(srcr_hbm.at[wid], sidx)
    pltpu.sync_copy(dstr_hbm.at[wid], didx)

    def body(j, carry):
        base = wid * ew + j * GCHUNK
        cp1 = pltpu.async_copy(h_hbm.at[didx.at[j]], bufh, sem1)
        cp2 = pltpu.async_copy(tp_hbm.at[didx.at[j]], buft, sem2)
        cp1.wait()
        cpa = pltpu.async_copy(h_hbm.at[sidx.at[j]], bufh, sem1, add=True)
        cp2.wait()
        for i in range(GCHUNK):
            tdbuf[i, :] = buft[i, pl.ds(0, 16)]
        cpa.wait()
        pltpu.sync_copy(bufh, hsum_hbm.at[pl.ds(base, GCHUNK)])
        pltpu.sync_copy(tdbuf, td_hbm.at[pl.ds(base, GCHUNK)])
        return carry

    lax.fori_loop(0, nj, body, 0)


def _sc_gather(h, tp, src, dst):
    E = src.shape[0]
    ew = E // NW
    nj = ew // GCHUNK
    srcr = src.reshape(NW, nj, GCHUNK)
    dstr = dst.reshape(NW, nj, GCHUNK)
    mesh = plsc.VectorSubcoreMesh(core_axis_name="c", subcore_axis_name="s")
    f = pl.kernel(
        _gather_body,
        out_type=[
            jax.ShapeDtypeStruct((E, 128), jnp.float32),
            jax.ShapeDtypeStruct((E, 16), jnp.float32),
        ],
        mesh=mesh,
        scratch_types=[
            pltpu.VMEM((nj, GCHUNK), jnp.int32),
            pltpu.VMEM((nj, GCHUNK), jnp.int32),
            pltpu.VMEM((GCHUNK, 128), jnp.float32),
            pltpu.VMEM((GCHUNK, 128), jnp.float32),
            pltpu.VMEM((GCHUNK, 16), jnp.float32),
            pltpu.SemaphoreType.DMA,
            pltpu.SemaphoreType.DMA,
        ],
    )
    return f(h, tp, srcr, dstr)


def _mlp_force_body(ef_ref, hsum_ref, td_ref, rx_ref, of_ref,
                    w1_ref, w2_ref, w3_ref,
                    b1_ref, g1_ref, be1_ref, b2_ref, g2_ref, be2_ref, b3_ref,
                    out_ref):
    feat = jnp.concatenate(
        [ef_ref[...], hsum_ref[...] * 0.5, td_ref[...]], axis=-1)
    x = jnp.dot(feat, w1_ref[...], preferred_element_type=jnp.float32)
    x = x + b1_ref[...]
    mu = jnp.mean(x, axis=-1, keepdims=True)
    xc = x - mu
    var = jnp.mean(xc * xc, axis=-1, keepdims=True)
    x = xc * jax.lax.rsqrt(var + 1e-5) * g1_ref[...] + be1_ref[...]
    x = jnp.maximum(x, 0.0)
    x = jnp.dot(x, w2_ref[...], preferred_element_type=jnp.float32)
    x = x + b2_ref[...]
    mu = jnp.mean(x, axis=-1, keepdims=True)
    xc = x - mu
    var = jnp.mean(xc * xc, axis=-1, keepdims=True)
    x = xc * jax.lax.rsqrt(var + 1e-5) * g2_ref[...] + be2_ref[...]
    x = jnp.maximum(x, 0.0)
    pred = jnp.sum(x * w3_ref[...], axis=-1, keepdims=True) + b3_ref[...]
    rx = rx_ref[...]
    d = jnp.sqrt(jnp.sum(rx * rx, axis=-1, keepdims=True))
    scale = pred * of_ref[...] / (d * (d + 1.0))
    out_ref[...] = rx * scale


def _mlp_force(ef, hsum, td, rx, of, W1, W2, w3r, b1r, g1r, be1r, b2r, g2r,
               be2r, b3r):
    E = ef.shape[0]
    grid = (E // E_BLOCK,)

    def eb(i):
        return (i, 0)

    def full(i):
        return (0, 0)

    return pl.pallas_call(
        _mlp_force_body,
        grid=grid,
        in_specs=[
            pl.BlockSpec((E_BLOCK, 16), eb),
            pl.BlockSpec((E_BLOCK, 128), eb),
            pl.BlockSpec((E_BLOCK, 16), eb),
            pl.BlockSpec((E_BLOCK, 3), eb),
            pl.BlockSpec((E_BLOCK, 1), eb),
            pl.BlockSpec((160, 256), full),
            pl.BlockSpec((256, 256), full),
            pl.BlockSpec((1, 256), full),
            pl.BlockSpec((1, 256), full),
            pl.BlockSpec((1, 256), full),
            pl.BlockSpec((1, 256), full),
            pl.BlockSpec((1, 256), full),
            pl.BlockSpec((1, 256), full),
            pl.BlockSpec((1, 256), full),
            pl.BlockSpec((1, 1), full),
        ],
        out_specs=pl.BlockSpec((E_BLOCK, 3), eb),
        out_shape=jax.ShapeDtypeStruct((E, 3), jnp.float32),
    )(ef, hsum, td, rx, of, W1, W2, w3r, b1r, g1r, be1r, b2r, g2r, be2r,
      b3r)


def kernel(h, rel_x, edge_feat, edge_index, inner_edge_mask, t,
           W1, b1, g1, beta1, W2, b2, g2, beta2, W3, b3):
    N = h.shape[0]
    E = rel_x.shape[0]
    src = edge_index[0].astype(jnp.int32)
    dst = edge_index[1].astype(jnp.int32)

    tp = jnp.pad(t, ((0, 0), (0, 112)))
    hsum, td = _sc_gather(h, tp, src, dst)

    of = (~inner_edge_mask).astype(jnp.float32).reshape(E, 1)
    w3r = W3.reshape(1, 256)
    b1r = b1.reshape(1, 256)
    g1r = g1.reshape(1, 256)
    be1r = beta1.reshape(1, 256)
    b2r = b2.reshape(1, 256)
    g2r = g2.reshape(1, 256)
    be2r = beta2.reshape(1, 256)
    b3r = b3.reshape(1, 1)

    forces = _mlp_force(edge_feat, hsum, td, rel_x, of, W1, W2, w3r, b1r,
                        g1r, be1r, b2r, g2r, be2r, b3r)

    out = jnp.zeros((N, 3), dtype=jnp.float32).at[dst].add(forces)
    return out


# speedup vs baseline: 2.0737x; 1.0281x over previous
import functools

import jax
import jax.numpy as jnp
from jax import lax
from jax.experimental import pallas as pl
from jax.experimental.pallas import tpu as pltpu
from jax.experimental.pallas import tpu_sc as plsc

E_BLOCK = 1280

NC, NS = 2, 16
NW = NC * NS
GCHUNK = 80


def _gather_body(h_hbm, tp_hbm, srcr_hbm, dstr_hbm, hsum_hbm, td_hbm,
                 sidx, didx, bufh, buft, tdbuf, sem1, sem2):
    ew = srcr_hbm.shape[1] * srcr_hbm.shape[2]
    nj = srcr_hbm.shape[1]
    wid = lax.axis_index("s") * NC + lax.axis_index("c")
    pltpu.sync_copy(srcr_hbm.at[wid], sidx)
    pltpu.sync_copy(dstr_hbm.at[wid], didx)

    def body(j, carry):
        base = wid * ew + j * GCHUNK
        cp1 = pltpu.async_copy(h_hbm.at[didx.at[j]], bufh, sem1)
        cp2 = pltpu.async_copy(tp_hbm.at[didx.at[j]], buft, sem2)
        cp1.wait()
        cpa = pltpu.async_copy(h_hbm.at[sidx.at[j]], bufh, sem1, add=True)
        cp2.wait()
        for i in range(GCHUNK):
            tdbuf[i, :] = buft[i, pl.ds(0, 16)]
        cpa.wait()
        pltpu.sync_copy(bufh, hsum_hbm.at[pl.ds(base, GCHUNK)])
        pltpu.sync_copy(tdbuf, td_hbm.at[pl.ds(base, GCHUNK)])
        return carry

    lax.fori_loop(0, nj, body, 0)


def _sc_gather(h, tp, src, dst):
    E = src.shape[0]
    ew = E // NW
    nj = ew // GCHUNK
    srcr = src.reshape(NW, nj, GCHUNK)
    dstr = dst.reshape(NW, nj, GCHUNK)
    mesh = plsc.VectorSubcoreMesh(core_axis_name="c", subcore_axis_name="s")
    f = pl.kernel(
        _gather_body,
        out_type=[
            jax.ShapeDtypeStruct((E, 128), jnp.float32),
            jax.ShapeDtypeStruct((E, 16), jnp.float32),
        ],
        mesh=mesh,
        scratch_types=[
            pltpu.VMEM((nj, GCHUNK), jnp.int32),
            pltpu.VMEM((nj, GCHUNK), jnp.int32),
            pltpu.VMEM((GCHUNK, 128), jnp.float32),
            pltpu.VMEM((GCHUNK, 128), jnp.float32),
            pltpu.VMEM((GCHUNK, 16), jnp.float32),
            pltpu.SemaphoreType.DMA,
            pltpu.SemaphoreType.DMA,
        ],
    )
    return f(h, tp, srcr, dstr)


def _mlp_force_body(ef_ref, hsum_ref, td_ref, rx_ref, of_ref,
                    w1_ref, w2_ref, w3_ref,
                    b1_ref, g1_ref, be1_ref, b2_ref, g2_ref, be2_ref, b3_ref,
                    out_ref):
    feat = jnp.concatenate(
        [ef_ref[...], hsum_ref[...] * 0.5, td_ref[...]], axis=-1)
    x = jnp.dot(feat.astype(jnp.bfloat16), w1_ref[...],
                preferred_element_type=jnp.float32)
    x = x + b1_ref[...]
    mu = jnp.mean(x, axis=-1, keepdims=True)
    xc = x - mu
    var = jnp.mean(xc * xc, axis=-1, keepdims=True)
    x = xc * jax.lax.rsqrt(var + 1e-5) * g1_ref[...] + be1_ref[...]
    x = jnp.maximum(x, 0.0)
    x = jnp.dot(x.astype(jnp.bfloat16), w2_ref[...],
                preferred_element_type=jnp.float32)
    x = x + b2_ref[...]
    mu = jnp.mean(x, axis=-1, keepdims=True)
    xc = x - mu
    var = jnp.mean(xc * xc, axis=-1, keepdims=True)
    x = xc * jax.lax.rsqrt(var + 1e-5) * g2_ref[...] + be2_ref[...]
    x = jnp.maximum(x, 0.0)
    pred = jnp.sum(x * w3_ref[...], axis=-1, keepdims=True) + b3_ref[...]
    rx = rx_ref[...]
    d = jnp.sqrt(jnp.sum(rx * rx, axis=-1, keepdims=True))
    scale = pred * of_ref[...] / (d * (d + 1.0))
    out_ref[...] = rx * scale


def _mlp_force(ef, hsum, td, rx, of, W1, W2, w3r, b1r, g1r, be1r, b2r, g2r,
               be2r, b3r):
    E = ef.shape[0]
    grid = (E // E_BLOCK,)

    def eb(i):
        return (i, 0)

    def full(i):
        return (0, 0)

    return pl.pallas_call(
        _mlp_force_body,
        grid=grid,
        in_specs=[
            pl.BlockSpec((E_BLOCK, 16), eb),
            pl.BlockSpec((E_BLOCK, 128), eb),
            pl.BlockSpec((E_BLOCK, 16), eb),
            pl.BlockSpec((E_BLOCK, 3), eb),
            pl.BlockSpec((E_BLOCK, 1), eb),
            pl.BlockSpec((160, 256), full),
            pl.BlockSpec((256, 256), full),
            pl.BlockSpec((1, 256), full),
            pl.BlockSpec((1, 256), full),
            pl.BlockSpec((1, 256), full),
            pl.BlockSpec((1, 256), full),
            pl.BlockSpec((1, 256), full),
            pl.BlockSpec((1, 256), full),
            pl.BlockSpec((1, 256), full),
            pl.BlockSpec((1, 1), full),
        ],
        out_specs=pl.BlockSpec((E_BLOCK, 3), eb),
        out_shape=jax.ShapeDtypeStruct((E, 3), jnp.float32),
    )(ef, hsum, td, rx, of, W1, W2, w3r, b1r, g1r, be1r, b2r, g2r, be2r,
      b3r)


def kernel(h, rel_x, edge_feat, edge_index, inner_edge_mask, t,
           W1, b1, g1, beta1, W2, b2, g2, beta2, W3, b3):
    N = h.shape[0]
    E = rel_x.shape[0]
    src = edge_index[0].astype(jnp.int32)
    dst = edge_index[1].astype(jnp.int32)

    tp = jnp.pad(t, ((0, 0), (0, 112)))
    hsum, td = _sc_gather(h, tp, src, dst)

    of = (~inner_edge_mask).astype(jnp.float32).reshape(E, 1)
    w3r = W3.reshape(1, 256)
    b1r = b1.reshape(1, 256)
    g1r = g1.reshape(1, 256)
    be1r = beta1.reshape(1, 256)
    b2r = b2.reshape(1, 256)
    g2r = g2.reshape(1, 256)
    be2r = beta2.reshape(1, 256)
    b3r = b3.reshape(1, 1)

    forces = _mlp_force(edge_feat, hsum, td, rel_x, of,
                        W1.astype(jnp.bfloat16), W2.astype(jnp.bfloat16),
                        w3r, b1r, g1r, be1r, b2r, g2r, be2r, b3r)

    out = jnp.zeros((N, 3), dtype=jnp.float32).at[dst].add(forces)
    return out
